# Initial kernel scaffold; baseline (speedup 1.0000x reference)
import jax
import jax.numpy as jnp
from jax.experimental import pallas as pl


def kernel(x, edge_index, etypes, Ws, bs, W_ih, W_hh, b_ih, b_hh):
    raise NotImplementedError("write your pallas kernel here")


# speedup vs baseline: 7.6143x; 7.6143x over previous
import jax
import jax.numpy as jnp
from jax import lax
from jax.experimental import pallas as pl
from jax.experimental.pallas import tpu as pltpu
from jax.experimental.pallas import tpu_sc as plsc

_N = 10000
_E = 320000
_D = 128
_T = 4
_STEPS = 2

_NC = 2
_NS = 16
_NW = _NC * _NS
_EPT = _E // _NW
_CB = 80
_SCH = 2000
_NSC = _EPT // _SCH
_NCH = _SCH // _CB
_NPAD = 10240
_RPS = _NPAD // _NS
_ZR = 64
_WBR = 128

_BN = 1000


def _sc_agg_body(y_hbm, src_hbm, et_hbm, dst_hbm, out_hbm,
                 src_v, et_v, gidx_v, dstf_v, dst_v, rows_v, zeros_v,
                 acc_sh, sem):
    c = lax.axis_index("c")
    s = lax.axis_index("s")
    wid = s * _NC + c

    zvec = jnp.zeros((16,), jnp.float32)

    def zrow(r, _):
        for k in range(_D // 16):
            zeros_v[r, pl.ds(k * 16, 16)] = zvec
        return 0
    lax.fori_loop(0, _ZR, zrow, 0)

    nbase = s * _RPS

    def zcp(k, _):
        pltpu.sync_copy(zeros_v, acc_sh.at[pl.ds(nbase + k * _ZR, _ZR)])
        return 0
    lax.fori_loop(0, _RPS // _ZR, zcp, 0)
    plsc.subcore_barrier()

    def superchunk(g, _):
        ebase = wid * _EPT + g * _SCH
        pltpu.sync_copy(src_hbm.at[pl.ds(ebase, _SCH)], src_v)
        pltpu.sync_copy(et_hbm.at[pl.ds(ebase, _SCH)], et_v)
        pltpu.sync_copy(dst_hbm.at[pl.ds(ebase, _SCH)], dstf_v)

        def pack(j, _):
            for k in range(_CB // 16):
                sl = pl.ds(j * _CB + k * 16, 16)
                gidx_v[sl] = et_v[sl] * _N + src_v[sl]
                dst_v[j, pl.ds(k * 16, 16)] = dstf_v[sl]
            return 0
        lax.fori_loop(0, _NCH, pack, 0)

        def chunk(j, _):
            pltpu.async_copy(y_hbm.at[gidx_v.at[pl.ds(j * _CB, _CB)]],
                             rows_v, sem).wait()
            pltpu.sync_copy(rows_v, acc_sh.at[dst_v.at[j]], add=True)
            return 0
        lax.fori_loop(0, _NCH, chunk, 0)
        return 0
    lax.fori_loop(0, _NSC, superchunk, 0)
    plsc.subcore_barrier()

    def wb(k, _):
        ro = nbase + k * _WBR
        pltpu.sync_copy(acc_sh.at[pl.ds(ro, _WBR)],
                        out_hbm.at[pl.ds(c * _NPAD + ro, _WBR)])
        return 0
    lax.fori_loop(0, _RPS // _WBR, wb, 0)


_sc_agg = pl.kernel(
    _sc_agg_body,
    out_type=jax.ShapeDtypeStruct((_NC * _NPAD, _D), jnp.float32),
    mesh=plsc.VectorSubcoreMesh(core_axis_name="c", subcore_axis_name="s"),
    scratch_types=[
        pltpu.VMEM((_SCH,), jnp.int32),
        pltpu.VMEM((_SCH,), jnp.int32),
        pltpu.VMEM((_SCH,), jnp.int32),
        pltpu.VMEM((_SCH,), jnp.int32),
        pltpu.VMEM((_NCH, _CB), jnp.int32),
        pltpu.VMEM((_CB, _D), jnp.float32),
        pltpu.VMEM((_ZR, _D), jnp.float32),
        pltpu.VMEM_SHARED((_NPAD, _D), jnp.float32),
        pltpu.SemaphoreType.DMA,
    ],
)


def _y_body(x_ref, w_ref, b_ref, y_ref):
    xb = x_ref[...]
    for i in range(_T):
        y_ref[i] = (jnp.dot(xb, w_ref[i], preferred_element_type=jnp.float32,
                            precision=lax.Precision.HIGHEST)
                    + b_ref[i][None, :])


_tc_y = pl.pallas_call(
    _y_body,
    grid=(_N // _BN,),
    in_specs=[
        pl.BlockSpec((_BN, _D), lambda j: (j, 0)),
        pl.BlockSpec((_T, _D, _D), lambda j: (0, 0, 0)),
        pl.BlockSpec((_T, _D), lambda j: (0, 0)),
    ],
    out_specs=pl.BlockSpec((_T, _BN, _D), lambda j: (0, j, 0)),
    out_shape=jax.ShapeDtypeStruct((_T, _N, _D), jnp.float32),
)


def _gru_body(p_ref, f_ref, wih_ref, whh_ref, bih_ref, bhh_ref, o_ref):
    a = p_ref[0] + p_ref[1]
    f = f_ref[...]
    gi = jnp.dot(a, wih_ref[...], preferred_element_type=jnp.float32,
                 precision=lax.Precision.HIGHEST) + bih_ref[...]
    gh = jnp.dot(f, whh_ref[...], preferred_element_type=jnp.float32,
                 precision=lax.Precision.HIGHEST) + bhh_ref[...]
    r = jax.nn.sigmoid(gi[:, :_D] + gh[:, :_D])
    z = jax.nn.sigmoid(gi[:, _D:2 * _D] + gh[:, _D:2 * _D])
    n = jnp.tanh(gi[:, 2 * _D:] + r * gh[:, 2 * _D:])
    o_ref[...] = (1.0 - z) * n + z * f


_tc_gru = pl.pallas_call(
    _gru_body,
    grid=(_N // _BN,),
    in_specs=[
        pl.BlockSpec((_NC, _BN, _D), lambda j: (0, j, 0)),
        pl.BlockSpec((_BN, _D), lambda j: (j, 0)),
        pl.BlockSpec((_D, 3 * _D), lambda j: (0, 0)),
        pl.BlockSpec((_D, 3 * _D), lambda j: (0, 0)),
        pl.BlockSpec((1, 3 * _D), lambda j: (0, 0)),
        pl.BlockSpec((1, 3 * _D), lambda j: (0, 0)),
    ],
    out_specs=pl.BlockSpec((_BN, _D), lambda j: (j, 0)),
    out_shape=jax.ShapeDtypeStruct((_N, _D), jnp.float32),
)


@jax.jit
def kernel(x, edge_index, etypes, Ws, bs, W_ih, W_hh, b_ih, b_hh):
    src = edge_index[0]
    dst = edge_index[1]

    WsT = Ws.transpose(0, 2, 1)
    W_ihT = W_ih.T
    W_hhT = W_hh.T
    b_ih2 = b_ih.reshape(1, 3 * _D)
    b_hh2 = b_hh.reshape(1, 3 * _D)

    feat = x
    for _ in range(_STEPS):
        y = _tc_y(feat, WsT, bs).reshape(_T * _N, _D)
        p = _sc_agg(y, src, etypes, dst).reshape(_NC, _NPAD, _D)[:, :_N]
        feat = _tc_gru(p, feat, W_ihT, W_hhT, b_ih2, b_hh2)
    return feat


# speedup vs baseline: 8.9062x; 1.1697x over previous
import jax
import jax.numpy as jnp
from jax import lax
from jax.experimental import pallas as pl
from jax.experimental.pallas import tpu as pltpu
from jax.experimental.pallas import tpu_sc as plsc

_N = 10000
_E = 320000
_D = 128
_T = 4
_STEPS = 2

_NC = 2
_NS = 16
_NW = _NC * _NS
_EPT = _E // _NW
_CB = 80
_SCH = 2000
_NSC = _EPT // _SCH
_NCH = _SCH // _CB
_NPAD = 10240
_RPS = _NPAD // _NS
_ZR = 64
_WBR = 128

_BN = 1000


def _sc_agg_body(y_hbm, src_hbm, et_hbm, dst_hbm, out_hbm,
                 src_v, et_v, gidx_v, dstf_v, dst_v, rows_v, zeros_v,
                 acc_sh, sem):
    c = lax.axis_index("c")
    s = lax.axis_index("s")
    wid = s * _NC + c

    zvec = jnp.zeros((16,), jnp.float32)

    def zrow(r, _):
        for k in range(_D // 16):
            zeros_v[r, pl.ds(k * 16, 16)] = zvec
        return 0
    lax.fori_loop(0, _ZR, zrow, 0)

    nbase = s * _RPS

    def zcp(k, _):
        pltpu.sync_copy(zeros_v, acc_sh.at[pl.ds(nbase + k * _ZR, _ZR)])
        return 0
    lax.fori_loop(0, _RPS // _ZR, zcp, 0)
    plsc.subcore_barrier()

    def superchunk(g, _):
        ebase = wid * _EPT + g * _SCH
        pltpu.sync_copy(src_hbm.at[pl.ds(ebase, _SCH)], src_v)
        pltpu.sync_copy(et_hbm.at[pl.ds(ebase, _SCH)], et_v)
        pltpu.sync_copy(dst_hbm.at[pl.ds(ebase, _SCH)], dstf_v)

        def pack(j, _):
            for k in range(_CB // 16):
                sl = pl.ds(j * _CB + k * 16, 16)
                gidx_v[sl] = et_v[sl] * _N + src_v[sl]
                dst_v[j, pl.ds(k * 16, 16)] = dstf_v[sl]
            return 0
        lax.fori_loop(0, _NCH, pack, 0)

        def chunk(j, _):
            pltpu.async_copy(y_hbm.at[gidx_v.at[pl.ds(j * _CB, _CB)]],
                             rows_v, sem).wait()
            pltpu.sync_copy(rows_v, acc_sh.at[dst_v.at[j]], add=True)
            return 0
        lax.fori_loop(0, _NCH, chunk, 0)
        return 0
    lax.fori_loop(0, _NSC, superchunk, 0)
    plsc.subcore_barrier()

    def wb(k, _):
        ro = nbase + k * _WBR
        pltpu.sync_copy(acc_sh.at[pl.ds(ro, _WBR)],
                        out_hbm.at[pl.ds(c * _NPAD + ro, _WBR)])
        return 0
    lax.fori_loop(0, _RPS // _WBR, wb, 0)


_sc_agg = pl.kernel(
    _sc_agg_body,
    out_type=jax.ShapeDtypeStruct((_NC * _NPAD, _D), jnp.float32),
    mesh=plsc.VectorSubcoreMesh(core_axis_name="c", subcore_axis_name="s"),
    scratch_types=[
        pltpu.VMEM((_SCH,), jnp.int32),
        pltpu.VMEM((_SCH,), jnp.int32),
        pltpu.VMEM((_SCH,), jnp.int32),
        pltpu.VMEM((_SCH,), jnp.int32),
        pltpu.VMEM((_NCH, _CB), jnp.int32),
        pltpu.VMEM((_CB, _D), jnp.float32),
        pltpu.VMEM((_ZR, _D), jnp.float32),
        pltpu.VMEM_SHARED((_NPAD, _D), jnp.float32),
        pltpu.SemaphoreType.DMA,
    ],
)


def _y_body(x_ref, w_ref, b_ref, y_ref):
    xb = x_ref[...]
    for i in range(_T):
        y_ref[i] = (jnp.dot(xb, w_ref[i], preferred_element_type=jnp.float32)
                    + b_ref[i][None, :])


_tc_y = pl.pallas_call(
    _y_body,
    grid=(_N // _BN,),
    in_specs=[
        pl.BlockSpec((_BN, _D), lambda j: (j, 0)),
        pl.BlockSpec((_T, _D, _D), lambda j: (0, 0, 0)),
        pl.BlockSpec((_T, _D), lambda j: (0, 0)),
    ],
    out_specs=pl.BlockSpec((_T, _BN, _D), lambda j: (0, j, 0)),
    out_shape=jax.ShapeDtypeStruct((_T, _N, _D), jnp.float32),
)


def _gru_body(p_ref, f_ref, wih_ref, whh_ref, bih_ref, bhh_ref, o_ref):
    a = p_ref[0] + p_ref[1]
    f = f_ref[...]
    gi = jnp.dot(a, wih_ref[...], preferred_element_type=jnp.float32) + bih_ref[...]
    gh = jnp.dot(f, whh_ref[...], preferred_element_type=jnp.float32) + bhh_ref[...]
    r = jax.nn.sigmoid(gi[:, :_D] + gh[:, :_D])
    z = jax.nn.sigmoid(gi[:, _D:2 * _D] + gh[:, _D:2 * _D])
    n = jnp.tanh(gi[:, 2 * _D:] + r * gh[:, 2 * _D:])
    o_ref[...] = (1.0 - z) * n + z * f


_tc_gru = pl.pallas_call(
    _gru_body,
    grid=(_N // _BN,),
    in_specs=[
        pl.BlockSpec((_NC, _BN, _D), lambda j: (0, j, 0)),
        pl.BlockSpec((_BN, _D), lambda j: (j, 0)),
        pl.BlockSpec((_D, 3 * _D), lambda j: (0, 0)),
        pl.BlockSpec((_D, 3 * _D), lambda j: (0, 0)),
        pl.BlockSpec((1, 3 * _D), lambda j: (0, 0)),
        pl.BlockSpec((1, 3 * _D), lambda j: (0, 0)),
    ],
    out_specs=pl.BlockSpec((_BN, _D), lambda j: (j, 0)),
    out_shape=jax.ShapeDtypeStruct((_N, _D), jnp.float32),
)


@jax.jit
def kernel(x, edge_index, etypes, Ws, bs, W_ih, W_hh, b_ih, b_hh):
    src = edge_index[0]
    dst = edge_index[1]

    WsT = Ws.transpose(0, 2, 1)
    W_ihT = W_ih.T
    W_hhT = W_hh.T
    b_ih2 = b_ih.reshape(1, 3 * _D)
    b_hh2 = b_hh.reshape(1, 3 * _D)

    feat = x
    for _ in range(_STEPS):
        y = _tc_y(feat, WsT, bs).reshape(_T * _N, _D)
        p = _sc_agg(y, src, etypes, dst).reshape(_NC, _NPAD, _D)[:, :_N]
        feat = _tc_gru(p, feat, W_ihT, W_hhT, b_ih2, b_hh2)
    return feat


# speedup vs baseline: 13.0230x; 1.4622x over previous
import jax
import jax.numpy as jnp
from jax import lax
from jax.experimental import pallas as pl
from jax.experimental.pallas import tpu as pltpu
from jax.experimental.pallas import tpu_sc as plsc

_N = 10000
_E = 320000
_D = 128
_T = 4
_STEPS = 2

_NC = 2
_NS = 16
_NW = _NC * _NS
_EPT = _E // _NW
_CB = 80
_SCH = 2000
_NSC = _EPT // _SCH
_NCH = _SCH // _CB
_NPAD = 10240
_RPS = _NPAD // _NS
_ZR = 64
_WBR = 128

_BN = 1000


def _sc_agg_body(y_hbm, src_hbm, et_hbm, dst_hbm, out_hbm,
                 src_v, et_v, gidx_v, dstf_v, dst_v, rows0_v, rows1_v,
                 zeros_v, acc_sh, sem0, sem1):
    c = lax.axis_index("c")
    s = lax.axis_index("s")
    wid = s * _NC + c

    zvec = jnp.zeros((16,), jnp.float32)

    def zrow(r, _):
        for k in range(_D // 16):
            zeros_v[r, pl.ds(k * 16, 16)] = zvec
        return 0
    lax.fori_loop(0, _ZR, zrow, 0)

    nbase = s * _RPS

    def zcp(k, _):
        pltpu.sync_copy(zeros_v, acc_sh.at[pl.ds(nbase + k * _ZR, _ZR)])
        return 0
    lax.fori_loop(0, _RPS // _ZR, zcp, 0)
    plsc.subcore_barrier()

    def superchunk(g, _):
        ebase = wid * _EPT + g * _SCH
        pltpu.sync_copy(src_hbm.at[pl.ds(ebase, _SCH)], src_v)
        pltpu.sync_copy(et_hbm.at[pl.ds(ebase, _SCH)], et_v)
        pltpu.sync_copy(dst_hbm.at[pl.ds(ebase, _SCH)], dstf_v)

        def pack(j, _):
            for k in range(_CB // 16):
                sl = pl.ds(j * _CB + k * 16, 16)
                gidx_v[sl] = et_v[sl] * _N + src_v[sl]
                dst_v[j, pl.ds(k * 16, 16)] = dstf_v[sl]
            return 0
        lax.fori_loop(0, _NCH, pack, 0)

        def fire(j, buf, sem_):
            pltpu.async_copy(y_hbm.at[gidx_v.at[pl.ds(j * _CB, _CB)]],
                             buf, sem_)

        def gwait(buf, sem_):
            pltpu.make_async_copy(y_hbm.at[gidx_v.at[pl.ds(0, _CB)]],
                                  buf, sem_).wait()

        def scat(j, buf):
            pltpu.sync_copy(buf, acc_sh.at[dst_v.at[j]], add=True)

        fire(0, rows0_v, sem0)

        def pipe(i, _):
            j = 2 * i
            fire(j + 1, rows1_v, sem1)
            gwait(rows0_v, sem0)
            scat(j, rows0_v)
            fire(j + 2, rows0_v, sem0)
            gwait(rows1_v, sem1)
            scat(j + 1, rows1_v)
            return 0
        lax.fori_loop(0, (_NCH - 1) // 2, pipe, 0)
        gwait(rows0_v, sem0)
        scat(_NCH - 1, rows0_v)
        return 0
    lax.fori_loop(0, _NSC, superchunk, 0)
    plsc.subcore_barrier()

    def wb(k, _):
        ro = nbase + k * _WBR
        pltpu.sync_copy(acc_sh.at[pl.ds(ro, _WBR)],
                        out_hbm.at[pl.ds(c * _NPAD + ro, _WBR)])
        return 0
    lax.fori_loop(0, _RPS // _WBR, wb, 0)


_sc_agg = pl.kernel(
    _sc_agg_body,
    out_type=jax.ShapeDtypeStruct((_NC * _NPAD, _D), jnp.float32),
    mesh=plsc.VectorSubcoreMesh(core_axis_name="c", subcore_axis_name="s"),
    scratch_types=[
        pltpu.VMEM((_SCH,), jnp.int32),
        pltpu.VMEM((_SCH,), jnp.int32),
        pltpu.VMEM((_SCH,), jnp.int32),
        pltpu.VMEM((_SCH,), jnp.int32),
        pltpu.VMEM((_NCH, _CB), jnp.int32),
        pltpu.VMEM((_CB, _D), jnp.float32),
        pltpu.VMEM((_CB, _D), jnp.float32),
        pltpu.VMEM((_ZR, _D), jnp.float32),
        pltpu.VMEM_SHARED((_NPAD, _D), jnp.float32),
        pltpu.SemaphoreType.DMA,
        pltpu.SemaphoreType.DMA,
    ],
)


def _y_body(x_ref, w_ref, b_ref, y_ref):
    xb = x_ref[...]
    for i in range(_T):
        y_ref[i] = (jnp.dot(xb, w_ref[i], preferred_element_type=jnp.float32)
                    + b_ref[i][None, :])


_tc_y = pl.pallas_call(
    _y_body,
    grid=(_N // _BN,),
    in_specs=[
        pl.BlockSpec((_BN, _D), lambda j: (j, 0)),
        pl.BlockSpec((_T, _D, _D), lambda j: (0, 0, 0)),
        pl.BlockSpec((_T, _D), lambda j: (0, 0)),
    ],
    out_specs=pl.BlockSpec((_T, _BN, _D), lambda j: (0, j, 0)),
    out_shape=jax.ShapeDtypeStruct((_T, _N, _D), jnp.float32),
)


def _gru_body(p_ref, f_ref, wih_ref, whh_ref, bih_ref, bhh_ref, o_ref):
    a = p_ref[0] + p_ref[1]
    f = f_ref[...]
    gi = jnp.dot(a, wih_ref[...], preferred_element_type=jnp.float32) + bih_ref[...]
    gh = jnp.dot(f, whh_ref[...], preferred_element_type=jnp.float32) + bhh_ref[...]
    r = jax.nn.sigmoid(gi[:, :_D] + gh[:, :_D])
    z = jax.nn.sigmoid(gi[:, _D:2 * _D] + gh[:, _D:2 * _D])
    n = jnp.tanh(gi[:, 2 * _D:] + r * gh[:, 2 * _D:])
    o_ref[...] = (1.0 - z) * n + z * f


_tc_gru = pl.pallas_call(
    _gru_body,
    grid=(_N // _BN,),
    in_specs=[
        pl.BlockSpec((_NC, _BN, _D), lambda j: (0, j, 0)),
        pl.BlockSpec((_BN, _D), lambda j: (j, 0)),
        pl.BlockSpec((_D, 3 * _D), lambda j: (0, 0)),
        pl.BlockSpec((_D, 3 * _D), lambda j: (0, 0)),
        pl.BlockSpec((1, 3 * _D), lambda j: (0, 0)),
        pl.BlockSpec((1, 3 * _D), lambda j: (0, 0)),
    ],
    out_specs=pl.BlockSpec((_BN, _D), lambda j: (j, 0)),
    out_shape=jax.ShapeDtypeStruct((_N, _D), jnp.float32),
)


@jax.jit
def kernel(x, edge_index, etypes, Ws, bs, W_ih, W_hh, b_ih, b_hh):
    src = edge_index[0]
    dst = edge_index[1]

    WsT = Ws.transpose(0, 2, 1)
    W_ihT = W_ih.T
    W_hhT = W_hh.T
    b_ih2 = b_ih.reshape(1, 3 * _D)
    b_hh2 = b_hh.reshape(1, 3 * _D)

    feat = x
    for _ in range(_STEPS):
        y = _tc_y(feat, WsT, bs).reshape(_T * _N, _D)
        p = _sc_agg(y, src, etypes, dst).reshape(_NC, _NPAD, _D)[:, :_N]
        feat = _tc_gru(p, feat, W_ihT, W_hhT, b_ih2, b_hh2)
    return feat


# speedup vs baseline: 13.5690x; 1.0419x over previous
import jax
import jax.numpy as jnp
from jax import lax
from jax.experimental import pallas as pl
from jax.experimental.pallas import tpu as pltpu
from jax.experimental.pallas import tpu_sc as plsc

_N = 10000
_E = 320000
_D = 128
_T = 4
_STEPS = 2

_NC = 2
_NS = 16
_NW = _NC * _NS
_EPT = _E // _NW
_CB = 80
_TCH = _EPT // _CB
_DPAD = 128
_NPAD = 10240
_RPS = _NPAD // _NS
_ZR = 64
_WBR = 128

_BN = 1000


def _sc_agg_body(y_hbm, gidx_hbm, dst2_hbm, zer_hbm, out_hbm,
                 gidx_v, dst_v, rows0_v, rows1_v,
                 acc_sh, sem0, sem1):
    c = lax.axis_index("c")
    s = lax.axis_index("s")
    wid = s * _NC + c
    nbase = s * _RPS

    pltpu.sync_copy(zer_hbm.at[pl.ds(nbase, _RPS)],
                    acc_sh.at[pl.ds(nbase, _RPS)])
    plsc.subcore_barrier()

    pltpu.sync_copy(gidx_hbm.at[pl.ds(wid * _EPT, _EPT)], gidx_v)
    pltpu.sync_copy(dst2_hbm.at[pl.ds(wid * _DPAD, _DPAD)], dst_v)

    def fire(j, buf, sem_):
        pltpu.async_copy(y_hbm.at[gidx_v.at[pl.ds(j * _CB, _CB)]],
                         buf, sem_)

    def gwait(buf, sem_):
        pltpu.make_async_copy(y_hbm.at[gidx_v.at[pl.ds(0, _CB)]],
                              buf, sem_).wait()

    def scat(j, buf):
        pltpu.sync_copy(buf, acc_sh.at[dst_v.at[j]], add=True)

    fire(0, rows0_v, sem0)

    def pipe(i, _):
        j = 2 * i
        fire(j + 1, rows1_v, sem1)
        gwait(rows0_v, sem0)
        scat(j, rows0_v)
        fire(j + 2, rows0_v, sem0)
        gwait(rows1_v, sem1)
        scat(j + 1, rows1_v)
        return 0
    lax.fori_loop(0, (_TCH - 1) // 2, pipe, 0)
    gwait(rows0_v, sem0)
    scat(_TCH - 1, rows0_v)
    plsc.subcore_barrier()

    def wb(k, _):
        ro = nbase + k * _WBR
        pltpu.sync_copy(acc_sh.at[pl.ds(ro, _WBR)],
                        out_hbm.at[pl.ds(c * _NPAD + ro, _WBR)])
        return 0
    lax.fori_loop(0, _RPS // _WBR, wb, 0)


_sc_agg = pl.kernel(
    _sc_agg_body,
    out_type=jax.ShapeDtypeStruct((_NC * _NPAD, _D), jnp.float32),
    mesh=plsc.VectorSubcoreMesh(core_axis_name="c", subcore_axis_name="s"),
    scratch_types=[
        pltpu.VMEM((_EPT,), jnp.int32),
        pltpu.VMEM((_DPAD, _CB), jnp.int32),
        pltpu.VMEM((_CB, _D), jnp.float32),
        pltpu.VMEM((_CB, _D), jnp.float32),
        pltpu.VMEM_SHARED((_NPAD, _D), jnp.float32),
        pltpu.SemaphoreType.DMA,
        pltpu.SemaphoreType.DMA,
    ],
)


def _gidx_body(s_ref, e_ref, o_ref):
    o_ref[...] = e_ref[...] * _N + s_ref[...]


_tc_gidx = pl.pallas_call(
    _gidx_body,
    out_shape=jax.ShapeDtypeStruct((_E // 128, 128), jnp.int32),
)


def _y_body(x_ref, w_ref, b_ref, y_ref):
    xb = x_ref[...]
    for i in range(_T):
        y_ref[i] = (jnp.dot(xb, w_ref[i], preferred_element_type=jnp.float32)
                    + b_ref[i][None, :])


_tc_y = pl.pallas_call(
    _y_body,
    grid=(_N // _BN,),
    in_specs=[
        pl.BlockSpec((_BN, _D), lambda j: (j, 0)),
        pl.BlockSpec((_T, _D, _D), lambda j: (0, 0, 0)),
        pl.BlockSpec((_T, _D), lambda j: (0, 0)),
    ],
    out_specs=pl.BlockSpec((_T, _BN, _D), lambda j: (0, j, 0)),
    out_shape=jax.ShapeDtypeStruct((_T, _N, _D), jnp.float32),
)


def _gru_body(p_ref, f_ref, wih_ref, whh_ref, bih_ref, bhh_ref, o_ref):
    a = p_ref[0] + p_ref[1]
    f = f_ref[...]
    gi = jnp.dot(a, wih_ref[...], preferred_element_type=jnp.float32) + bih_ref[...]
    gh = jnp.dot(f, whh_ref[...], preferred_element_type=jnp.float32) + bhh_ref[...]
    r = jax.nn.sigmoid(gi[:, :_D] + gh[:, :_D])
    z = jax.nn.sigmoid(gi[:, _D:2 * _D] + gh[:, _D:2 * _D])
    n = jnp.tanh(gi[:, 2 * _D:] + r * gh[:, 2 * _D:])
    o_ref[...] = (1.0 - z) * n + z * f


_tc_gru = pl.pallas_call(
    _gru_body,
    grid=(_N // _BN,),
    in_specs=[
        pl.BlockSpec((_NC, _BN, _D), lambda j: (0, j, 0)),
        pl.BlockSpec((_BN, _D), lambda j: (j, 0)),
        pl.BlockSpec((_D, 3 * _D), lambda j: (0, 0)),
        pl.BlockSpec((_D, 3 * _D), lambda j: (0, 0)),
        pl.BlockSpec((1, 3 * _D), lambda j: (0, 0)),
        pl.BlockSpec((1, 3 * _D), lambda j: (0, 0)),
    ],
    out_specs=pl.BlockSpec((_BN, _D), lambda j: (j, 0)),
    out_shape=jax.ShapeDtypeStruct((_N, _D), jnp.float32),
)


@jax.jit
def kernel(x, edge_index, etypes, Ws, bs, W_ih, W_hh, b_ih, b_hh):
    src = edge_index[0]
    dst = edge_index[1]

    WsT = Ws.transpose(0, 2, 1)
    W_ihT = W_ih.T
    W_hhT = W_hh.T
    b_ih2 = b_ih.reshape(1, 3 * _D)
    b_hh2 = b_hh.reshape(1, 3 * _D)

    gidx = _tc_gidx(src.reshape(_E // 128, 128),
                    etypes.reshape(_E // 128, 128)).reshape(_E)
    dst2 = jnp.pad(dst.reshape(_NW, _TCH, _CB),
                   ((0, 0), (0, _DPAD - _TCH), (0, 0))).reshape(
                       _NW * _DPAD, _CB)

    zer = jnp.zeros((_NPAD, _D), jnp.float32)
    feat = x
    for _ in range(_STEPS):
        y = _tc_y(feat, WsT, bs).reshape(_T * _N, _D)
        p = _sc_agg(y, gidx, dst2, zer).reshape(_NC, _NPAD, _D)[:, :_N]
        feat = _tc_gru(p, feat, W_ihT, W_hhT, b_ih2, b_hh2)
    return feat


# speedup vs baseline: 14.1362x; 1.0418x over previous
import jax
import jax.numpy as jnp
from jax import lax
from jax.experimental import pallas as pl
from jax.experimental.pallas import tpu as pltpu
from jax.experimental.pallas import tpu_sc as plsc

_N = 10000
_E = 320000
_D = 128
_T = 4
_STEPS = 2

_NC = 2
_NS = 16
_NW = _NC * _NS
_EPT = _E // _NW
_CB = 80
_TCH = _EPT // _CB
_DPAD = 128
_NPAD = 10240
_RPS = _NPAD // _NS
_ZR = 64
_WBR = 128

_BN = 1024


def _sc_agg_body(y_hbm, gidx_hbm, dst2_hbm, zer_hbm, out_hbm,
                 gidx_v, dst_v, rows0_v, rows1_v,
                 acc_sh, sem0, sem1):
    c = lax.axis_index("c")
    s = lax.axis_index("s")
    wid = s * _NC + c
    nbase = s * _RPS

    pltpu.sync_copy(zer_hbm.at[pl.ds(nbase, _RPS)],
                    acc_sh.at[pl.ds(nbase, _RPS)])
    plsc.subcore_barrier()

    pltpu.sync_copy(gidx_hbm.at[pl.ds(wid * _EPT, _EPT)], gidx_v)
    pltpu.sync_copy(dst2_hbm.at[pl.ds(wid * _DPAD, _DPAD)], dst_v)

    def fire(j, buf, sem_):
        pltpu.async_copy(y_hbm.at[gidx_v.at[pl.ds(j * _CB, _CB)]],
                         buf, sem_)

    def gwait(buf, sem_):
        pltpu.make_async_copy(y_hbm.at[gidx_v.at[pl.ds(0, _CB)]],
                              buf, sem_).wait()

    def scat(j, buf):
        pltpu.sync_copy(buf, acc_sh.at[dst_v.at[j]], add=True)

    fire(0, rows0_v, sem0)

    def pipe(i, _):
        j = 2 * i
        fire(j + 1, rows1_v, sem1)
        gwait(rows0_v, sem0)
        scat(j, rows0_v)
        fire(j + 2, rows0_v, sem0)
        gwait(rows1_v, sem1)
        scat(j + 1, rows1_v)
        return 0
    lax.fori_loop(0, (_TCH - 1) // 2, pipe, 0)
    gwait(rows0_v, sem0)
    scat(_TCH - 1, rows0_v)
    plsc.subcore_barrier()

    def wb(k, _):
        ro = nbase + k * _WBR
        pltpu.sync_copy(acc_sh.at[pl.ds(ro, _WBR)],
                        out_hbm.at[pl.ds(c * _NPAD + ro, _WBR)])
        return 0
    lax.fori_loop(0, _RPS // _WBR, wb, 0)


_sc_agg = pl.kernel(
    _sc_agg_body,
    out_type=jax.ShapeDtypeStruct((_NC * _NPAD, _D), jnp.float32),
    mesh=plsc.VectorSubcoreMesh(core_axis_name="c", subcore_axis_name="s"),
    scratch_types=[
        pltpu.VMEM((_EPT,), jnp.int32),
        pltpu.VMEM((_DPAD, _CB), jnp.int32),
        pltpu.VMEM((_CB, _D), jnp.float32),
        pltpu.VMEM((_CB, _D), jnp.float32),
        pltpu.VMEM_SHARED((_NPAD, _D), jnp.float32),
        pltpu.SemaphoreType.DMA,
        pltpu.SemaphoreType.DMA,
    ],
)


def _gidx_body(s_ref, e_ref, o_ref):
    o_ref[...] = e_ref[...] * _NPAD + s_ref[...]


_tc_gidx = pl.pallas_call(
    _gidx_body,
    out_shape=jax.ShapeDtypeStruct((_E // 128, 128), jnp.int32),
)


def _y_body(x_ref, w_ref, b_ref, y_ref):
    xb = x_ref[...]
    for i in range(_T):
        y_ref[i] = (jnp.dot(xb, w_ref[i], preferred_element_type=jnp.float32)
                    + b_ref[i][None, :])


_tc_y = pl.pallas_call(
    _y_body,
    grid=(_NPAD // _BN,),
    in_specs=[
        pl.BlockSpec((_BN, _D), lambda j: (j, 0)),
        pl.BlockSpec((_T, _D, _D), lambda j: (0, 0, 0)),
        pl.BlockSpec((_T, _D), lambda j: (0, 0)),
    ],
    out_specs=pl.BlockSpec((_T, _BN, _D), lambda j: (0, j, 0)),
    out_shape=jax.ShapeDtypeStruct((_T, _NPAD, _D), jnp.float32),
)


def _gru_math(p_ref, f_ref, wih_ref, whh_ref, bih_ref, bhh_ref):
    a = p_ref[0] + p_ref[1]
    f = f_ref[...]
    gi = jnp.dot(a, wih_ref[...], preferred_element_type=jnp.float32) + bih_ref[...]
    gh = jnp.dot(f, whh_ref[...], preferred_element_type=jnp.float32) + bhh_ref[...]
    r = jax.nn.sigmoid(gi[:, :_D] + gh[:, :_D])
    z = jax.nn.sigmoid(gi[:, _D:2 * _D] + gh[:, _D:2 * _D])
    n = jnp.tanh(gi[:, 2 * _D:] + r * gh[:, 2 * _D:])
    return (1.0 - z) * n + z * f


def _gru_body(p_ref, f_ref, wih_ref, whh_ref, bih_ref, bhh_ref, o_ref):
    o_ref[...] = _gru_math(p_ref, f_ref, wih_ref, whh_ref, bih_ref, bhh_ref)


def _gru_y_body(p_ref, f_ref, wih_ref, whh_ref, bih_ref, bhh_ref, w_ref,
                b_ref, o_ref, y_ref):
    fn = _gru_math(p_ref, f_ref, wih_ref, whh_ref, bih_ref, bhh_ref)
    o_ref[...] = fn
    for i in range(_T):
        y_ref[i] = (jnp.dot(fn, w_ref[i], preferred_element_type=jnp.float32)
                    + b_ref[i][None, :])


_gru_in_specs = [
    pl.BlockSpec((_NC, _BN, _D), lambda j: (0, j, 0)),
    pl.BlockSpec((_BN, _D), lambda j: (j, 0)),
    pl.BlockSpec((_D, 3 * _D), lambda j: (0, 0)),
    pl.BlockSpec((_D, 3 * _D), lambda j: (0, 0)),
    pl.BlockSpec((1, 3 * _D), lambda j: (0, 0)),
    pl.BlockSpec((1, 3 * _D), lambda j: (0, 0)),
]

_tc_gru = pl.pallas_call(
    _gru_body,
    grid=(_NPAD // _BN,),
    in_specs=_gru_in_specs,
    out_specs=pl.BlockSpec((_BN, _D), lambda j: (j, 0)),
    out_shape=jax.ShapeDtypeStruct((_NPAD, _D), jnp.float32),
)

_tc_gru_y = pl.pallas_call(
    _gru_y_body,
    grid=(_NPAD // _BN,),
    in_specs=_gru_in_specs + [
        pl.BlockSpec((_T, _D, _D), lambda j: (0, 0, 0)),
        pl.BlockSpec((_T, _D), lambda j: (0, 0)),
    ],
    out_specs=[
        pl.BlockSpec((_BN, _D), lambda j: (j, 0)),
        pl.BlockSpec((_T, _BN, _D), lambda j: (0, j, 0)),
    ],
    out_shape=[
        jax.ShapeDtypeStruct((_NPAD, _D), jnp.float32),
        jax.ShapeDtypeStruct((_T, _NPAD, _D), jnp.float32),
    ],
)


@jax.jit
def kernel(x, edge_index, etypes, Ws, bs, W_ih, W_hh, b_ih, b_hh):
    src = edge_index[0]
    dst = edge_index[1]

    WsT = Ws.transpose(0, 2, 1)
    W_ihT = W_ih.T
    W_hhT = W_hh.T
    b_ih2 = b_ih.reshape(1, 3 * _D)
    b_hh2 = b_hh.reshape(1, 3 * _D)

    gidx = _tc_gidx(src.reshape(_E // 128, 128),
                    etypes.reshape(_E // 128, 128)).reshape(_E)
    dst2 = jnp.pad(dst.reshape(_NW, _TCH, _CB),
                   ((0, 0), (0, _DPAD - _TCH), (0, 0))).reshape(
                       _NW * _DPAD, _CB)

    zer = jnp.zeros((_NPAD, _D), jnp.float32)
    feat = jnp.pad(x, ((0, _NPAD - _N), (0, 0)))
    y = _tc_y(feat, WsT, bs).reshape(_T * _NPAD, _D)
    p = _sc_agg(y, gidx, dst2, zer).reshape(_NC, _NPAD, _D)
    feat, y = _tc_gru_y(p, feat, W_ihT, W_hhT, b_ih2, b_hh2, WsT, bs)
    p = _sc_agg(y.reshape(_T * _NPAD, _D), gidx, dst2, zer).reshape(
        _NC, _NPAD, _D)
    feat = _tc_gru(p, feat, W_ihT, W_hhT, b_ih2, b_hh2)
    return feat[:_N]


# speedup vs baseline: 14.2666x; 1.0092x over previous
import jax
import jax.numpy as jnp
from jax import lax
from jax.experimental import pallas as pl
from jax.experimental.pallas import tpu as pltpu
from jax.experimental.pallas import tpu_sc as plsc

_N = 10000
_E = 320000
_D = 128
_T = 4
_STEPS = 2

_NC = 2
_NS = 16
_NW = _NC * _NS
_EPT = _E // _NW
_CB = 80
_TCH = _EPT // _CB
_DPAD = 128
_NPAD = 10240
_RPS = _NPAD // _NS
_ZR = 64
_WBR = 128

_BN = 1024


def _sc_agg_body(y_hbm, gidx_hbm, dst2_hbm, zer_hbm, out_hbm,
                 gidx_v, dst_v, rows0_v, rows1_v,
                 acc_sh, sem0, sem1):
    c = lax.axis_index("c")
    s = lax.axis_index("s")
    wid = s * _NC + c
    nbase = s * _RPS

    pltpu.sync_copy(gidx_hbm.at[pl.ds(wid * _EPT, _EPT)], gidx_v)

    def fire(j, buf, sem_):
        pltpu.async_copy(y_hbm.at[gidx_v.at[pl.ds(j * _CB, _CB)]],
                         buf, sem_)

    def gwait(buf, sem_):
        pltpu.make_async_copy(y_hbm.at[gidx_v.at[pl.ds(0, _CB)]],
                              buf, sem_).wait()

    def scat(j, buf):
        pltpu.sync_copy(buf, acc_sh.at[dst_v.at[j]], add=True)

    fire(0, rows0_v, sem0)
    pltpu.sync_copy(dst2_hbm.at[pl.ds(wid * _DPAD, _DPAD)], dst_v)
    fire(1, rows1_v, sem1)
    pltpu.sync_copy(zer_hbm.at[pl.ds(nbase, _RPS)],
                    acc_sh.at[pl.ds(nbase, _RPS)])
    plsc.subcore_barrier()

    def pipe(i, _):
        j = 2 * i
        gwait(rows0_v, sem0)
        scat(j, rows0_v)
        fire(j + 2, rows0_v, sem0)
        gwait(rows1_v, sem1)
        scat(j + 1, rows1_v)
        fire(j + 3, rows1_v, sem1)
        return 0
    lax.fori_loop(0, (_TCH - 3) // 2, pipe, 0)
    gwait(rows0_v, sem0)
    scat(_TCH - 3, rows0_v)
    fire(_TCH - 1, rows0_v, sem0)
    gwait(rows1_v, sem1)
    scat(_TCH - 2, rows1_v)
    gwait(rows0_v, sem0)
    scat(_TCH - 1, rows0_v)
    plsc.subcore_barrier()

    def wb(k, _):
        ro = nbase + k * _WBR
        pltpu.sync_copy(acc_sh.at[pl.ds(ro, _WBR)],
                        out_hbm.at[pl.ds(c * _NPAD + ro, _WBR)])
        return 0
    lax.fori_loop(0, _RPS // _WBR, wb, 0)


_sc_agg = pl.kernel(
    _sc_agg_body,
    out_type=jax.ShapeDtypeStruct((_NC * _NPAD, _D), jnp.float32),
    mesh=plsc.VectorSubcoreMesh(core_axis_name="c", subcore_axis_name="s"),
    scratch_types=[
        pltpu.VMEM((_EPT,), jnp.int32),
        pltpu.VMEM((_DPAD, _CB), jnp.int32),
        pltpu.VMEM((_CB, _D), jnp.float32),
        pltpu.VMEM((_CB, _D), jnp.float32),
        pltpu.VMEM_SHARED((_NPAD, _D), jnp.float32),
        pltpu.SemaphoreType.DMA,
        pltpu.SemaphoreType.DMA,
    ],
)


def _gidx_body(s_ref, e_ref, o_ref):
    o_ref[...] = e_ref[...] * _NPAD + s_ref[...]


_tc_gidx = pl.pallas_call(
    _gidx_body,
    out_shape=jax.ShapeDtypeStruct((_E // 128, 128), jnp.int32),
)


def _y_body(x_ref, w_ref, b_ref, y_ref):
    xb = x_ref[...]
    for i in range(_T):
        y_ref[i] = (jnp.dot(xb, w_ref[i], preferred_element_type=jnp.float32)
                    + b_ref[i][None, :])


_tc_y = pl.pallas_call(
    _y_body,
    grid=(_NPAD // _BN,),
    in_specs=[
        pl.BlockSpec((_BN, _D), lambda j: (j, 0)),
        pl.BlockSpec((_T, _D, _D), lambda j: (0, 0, 0)),
        pl.BlockSpec((_T, _D), lambda j: (0, 0)),
    ],
    out_specs=pl.BlockSpec((_T, _BN, _D), lambda j: (0, j, 0)),
    out_shape=jax.ShapeDtypeStruct((_T, _NPAD, _D), jnp.float32),
)


def _gru_math(p_ref, f_ref, wih_ref, whh_ref, bih_ref, bhh_ref):
    a = p_ref[0] + p_ref[1]
    f = f_ref[...]
    gi = jnp.dot(a, wih_ref[...], preferred_element_type=jnp.float32) + bih_ref[...]
    gh = jnp.dot(f, whh_ref[...], preferred_element_type=jnp.float32) + bhh_ref[...]
    r = jax.nn.sigmoid(gi[:, :_D] + gh[:, :_D])
    z = jax.nn.sigmoid(gi[:, _D:2 * _D] + gh[:, _D:2 * _D])
    n = jnp.tanh(gi[:, 2 * _D:] + r * gh[:, 2 * _D:])
    return (1.0 - z) * n + z * f


def _gru_body(p_ref, f_ref, wih_ref, whh_ref, bih_ref, bhh_ref, o_ref):
    o_ref[...] = _gru_math(p_ref, f_ref, wih_ref, whh_ref, bih_ref, bhh_ref)


def _gru_y_body(p_ref, f_ref, wih_ref, whh_ref, bih_ref, bhh_ref, w_ref,
                b_ref, o_ref, y_ref):
    fn = _gru_math(p_ref, f_ref, wih_ref, whh_ref, bih_ref, bhh_ref)
    o_ref[...] = fn
    for i in range(_T):
        y_ref[i] = (jnp.dot(fn, w_ref[i], preferred_element_type=jnp.float32)
                    + b_ref[i][None, :])


_gru_in_specs = [
    pl.BlockSpec((_NC, _BN, _D), lambda j: (0, j, 0)),
    pl.BlockSpec((_BN, _D), lambda j: (j, 0)),
    pl.BlockSpec((_D, 3 * _D), lambda j: (0, 0)),
    pl.BlockSpec((_D, 3 * _D), lambda j: (0, 0)),
    pl.BlockSpec((1, 3 * _D), lambda j: (0, 0)),
    pl.BlockSpec((1, 3 * _D), lambda j: (0, 0)),
]

_tc_gru = pl.pallas_call(
    _gru_body,
    grid=(_NPAD // _BN,),
    in_specs=_gru_in_specs,
    out_specs=pl.BlockSpec((_BN, _D), lambda j: (j, 0)),
    out_shape=jax.ShapeDtypeStruct((_NPAD, _D), jnp.float32),
)

_tc_gru_y = pl.pallas_call(
    _gru_y_body,
    grid=(_NPAD // _BN,),
    in_specs=_gru_in_specs + [
        pl.BlockSpec((_T, _D, _D), lambda j: (0, 0, 0)),
        pl.BlockSpec((_T, _D), lambda j: (0, 0)),
    ],
    out_specs=[
        pl.BlockSpec((_BN, _D), lambda j: (j, 0)),
        pl.BlockSpec((_T, _BN, _D), lambda j: (0, j, 0)),
    ],
    out_shape=[
        jax.ShapeDtypeStruct((_NPAD, _D), jnp.float32),
        jax.ShapeDtypeStruct((_T, _NPAD, _D), jnp.float32),
    ],
)


@jax.jit
def kernel(x, edge_index, etypes, Ws, bs, W_ih, W_hh, b_ih, b_hh):
    src = edge_index[0]
    dst = edge_index[1]

    WsT = Ws.transpose(0, 2, 1)
    W_ihT = W_ih.T
    W_hhT = W_hh.T
    b_ih2 = b_ih.reshape(1, 3 * _D)
    b_hh2 = b_hh.reshape(1, 3 * _D)

    dst2 = jnp.pad(dst.reshape(_NW, _TCH, _CB),
                   ((0, 0), (0, _DPAD - _TCH), (0, 0))).reshape(
                       _NW * _DPAD, _CB)
    zer = jnp.zeros((_NPAD, _D), jnp.float32)
    gidx = _tc_gidx(src.reshape(_E // 128, 128),
                    etypes.reshape(_E // 128, 128)).reshape(_E)

    feat = jnp.pad(x, ((0, _NPAD - _N), (0, 0)))
    y = _tc_y(feat, WsT, bs)
    p = _sc_agg(y.reshape(_T * _NPAD, _D), gidx, dst2, zer).reshape(
        _NC, _NPAD, _D)
    feat, y = _tc_gru_y(p, feat, W_ihT, W_hhT, b_ih2, b_hh2, WsT, bs)
    p = _sc_agg(y.reshape(_T * _NPAD, _D), gidx, dst2, zer).reshape(
        _NC, _NPAD, _D)
    feat = _tc_gru(p, feat, W_ihT, W_hhT, b_ih2, b_hh2)
    return feat[:_N]
